# speedup vs baseline: 1.0214x; 1.0214x over previous
import jax
import jax.numpy as jnp
from jax.experimental import pallas as pl
from jax.experimental.pallas import tpu as pltpu

BATCH_BLOCK = 8


def _add_kernel(patch_ref, pos_ref, out_ref):
    out_ref[...] = patch_ref[...] + pos_ref[...][None, :, :]


def kernel(patch, pos_table):
    batch, num_patches, proj_dim = patch.shape
    grid = (batch // BATCH_BLOCK,)
    return pl.pallas_call(
        _add_kernel,
        grid=grid,
        in_specs=[
            pl.BlockSpec((BATCH_BLOCK, num_patches, proj_dim), lambda i: (i, 0, 0)),
            pl.BlockSpec((num_patches, proj_dim), lambda i: (0, 0)),
        ],
        out_specs=pl.BlockSpec((BATCH_BLOCK, num_patches, proj_dim), lambda i: (i, 0, 0)),
        out_shape=jax.ShapeDtypeStruct(patch.shape, patch.dtype),
    )(patch, pos_table)
